# initial kernel scaffold (unmeasured)
import jax
import jax.numpy as jnp
from jax import lax
from jax.experimental import pallas as pl
from jax.experimental.pallas import tpu as pltpu

B, H, D, BS = 8, 8, 128, 16
NP_LOCAL = 512
CP = 64
NC = NP_LOCAL // CP
T = CP * BS
NEG = -1e30


def kernel(Q, K, V, bt, lens):
    my_x = lax.axis_index("x")
    my_y = lax.axis_index("y")
    my_z = lax.axis_index("z")
    lo = my_x * NP_LOCAL

    nb = bt.shape[1]
    valid = (jnp.arange(nb)[None, :] < lens[:, None]) & (bt >= lo) & (bt < lo + NP_LOCAL)
    rel = bt - lo
    onehot = (rel[:, :, None] == jnp.arange(NP_LOCAL)[None, None, :]) & valid[:, :, None]
    counts = onehot.sum(axis=1).astype(jnp.float32)
    cnt_tok = jnp.repeat(counts, BS, axis=1)

    def body(q_ref, k_ref, v_ref, c_ref, out_ref,
             acc_s, m_s, l_s, comm, send_sem, recv_sem):
        c = pl.program_id(0)

        @pl.when(c == 0)
        def _():
            barrier = pltpu.get_barrier_semaphore()
            pl.semaphore_signal(
                barrier, inc=1,
                device_id=(1 - my_x, my_y, my_z),
                device_id_type=pl.DeviceIdType.MESH,
            )
            pl.semaphore_wait(barrier, 1)
            m_s[...] = jnp.full((H, B), NEG, jnp.float32)
            l_s[...] = jnp.zeros((H, B), jnp.float32)
            acc_s[...] = jnp.zeros((H, B, D), jnp.float32)

        q = q_ref[:, 0, :, :].astype(jnp.bfloat16)
        k = k_ref[...].reshape(T, H, D).astype(jnp.bfloat16)
        v = v_ref[...].reshape(T, H, D).astype(jnp.bfloat16)
        cnt = c_ref[...]

        s = lax.dot_general(
            q, k, (((2,), (2,)), ((1,), (1,))),
            preferred_element_type=jnp.float32,
        ) * (D ** -0.5)
        s = jnp.where((cnt > 0.0)[None, :, :], s, NEG)

        m_prev = m_s[...]
        m_new = jnp.maximum(m_prev, jnp.max(s, axis=-1))
        alpha = jnp.exp(m_prev - m_new)
        p = jnp.exp(s - m_new[:, :, None]) * cnt[None, :, :]
        l_s[...] = l_s[...] * alpha + jnp.sum(p, axis=-1)
        pv = lax.dot_general(
            p.astype(jnp.bfloat16), v, (((2,), (0,)), ((0,), (1,))),
            preferred_element_type=jnp.float32,
        )
        acc_s[...] = acc_s[...] * alpha[:, :, None] + pv
        m_s[...] = m_new

        @pl.when(c == NC - 1)
        def _():
            comm[0, 0:H, :, :] = acc_s[...]
            comm[0, H, :, 0:B] = m_s[...]
            comm[0, H + 1, :, 0:B] = l_s[...]

            rdma = pltpu.make_async_remote_copy(
                src_ref=comm.at[0],
                dst_ref=comm.at[1],
                send_sem=send_sem,
                recv_sem=recv_sem,
                device_id=(1 - my_x, my_y, my_z),
                device_id_type=pl.DeviceIdType.MESH,
            )
            rdma.start()
            rdma.wait()

            acc1 = comm[1, 0:H, :, :]
            m1 = comm[1, H, :, 0:B]
            l1 = comm[1, H + 1, :, 0:B]
            m0, l0, acc0 = m_s[...], l_s[...], acc_s[...]
            m_all = jnp.maximum(m0, m1)
            a0 = jnp.exp(m0 - m_all)
            a1 = jnp.exp(m1 - m_all)
            l_all = a0 * l0 + a1 * l1
            acc = a0[:, :, None] * acc0 + a1[:, :, None] * acc1
            o = acc / l_all[:, :, None]
            for h in range(H):
                out_ref[:, 0, h, :] = o[h, :, :]

    return pl.pallas_call(
        body,
        grid=(NC,),
        in_specs=[
            pl.BlockSpec((B, 1, H, D), lambda c: (0, 0, 0, 0)),
            pl.BlockSpec((CP, BS, H, D), lambda c: (c, 0, 0, 0)),
            pl.BlockSpec((CP, BS, H, D), lambda c: (c, 0, 0, 0)),
            pl.BlockSpec((B, T), lambda c: (0, c)),
        ],
        out_specs=pl.BlockSpec((B, 1, H, D), lambda c: (0, 0, 0, 0)),
        out_shape=jax.ShapeDtypeStruct((B, 1, H, D), jnp.float32),
        scratch_shapes=[
            pltpu.VMEM((H, B, D), jnp.float32),
            pltpu.VMEM((H, B), jnp.float32),
            pltpu.VMEM((H, B), jnp.float32),
            pltpu.VMEM((2, H + 2, B, D), jnp.float32),
            pltpu.SemaphoreType.DMA,
            pltpu.SemaphoreType.DMA,
        ],
        compiler_params=pltpu.CompilerParams(collective_id=0),
    )(Q, K, V, cnt_tok)


# baseline (device time: 171270 ns/iter reference)
import jax
import jax.numpy as jnp
from jax import lax
from jax.experimental import pallas as pl
from jax.experimental.pallas import tpu as pltpu

B, H, D, BS = 8, 8, 128, 16
NP_LOCAL = 512
CP = 32
NC = NP_LOCAL // CP
T = CP * BS
NEG = -1e30


def kernel(Q, K, V, bt, lens):
    my_x = lax.axis_index("x")
    my_y = lax.axis_index("y")
    my_z = lax.axis_index("z")
    lo = my_x * NP_LOCAL

    nb = bt.shape[1]
    valid = (jnp.arange(nb)[None, :] < lens[:, None]) & (bt >= lo) & (bt < lo + NP_LOCAL)
    rel = bt - lo
    onehot = (rel[:, :, None] == jnp.arange(NP_LOCAL)[None, None, :]) & valid[:, :, None]
    counts = onehot.sum(axis=1).astype(jnp.float32)
    cnt_tok = jnp.repeat(counts, BS, axis=1)

    def body(q_ref, k_ref, v_ref, c_ref, out_ref,
             acc_s, m_s, l_s, comm, send_sem, recv_sem):
        c = pl.program_id(0)
        my_x = lax.axis_index("x")
        my_y = lax.axis_index("y")
        my_z = lax.axis_index("z")

        @pl.when(c == 0)
        def _():
            barrier = pltpu.get_barrier_semaphore()
            pl.semaphore_signal(
                barrier, inc=1,
                device_id=(1 - my_x, my_y, my_z),
                device_id_type=pl.DeviceIdType.MESH,
            )
            pl.semaphore_wait(barrier, 1)
            m_s[...] = jnp.full((H, B), NEG, jnp.float32)
            l_s[...] = jnp.zeros((H, B), jnp.float32)
            acc_s[...] = jnp.zeros((H, B, D), jnp.float32)

        q = q_ref[:, 0, :, :].astype(jnp.bfloat16)
        k = k_ref[...].reshape(T, H, D).astype(jnp.bfloat16)
        v = v_ref[...].reshape(T, H, D).astype(jnp.bfloat16)
        cnt = c_ref[...]

        s = lax.dot_general(
            q, k, (((2,), (2,)), ((1,), (1,))),
            preferred_element_type=jnp.float32,
        ) * (D ** -0.5)
        s = jnp.where((cnt > 0.0)[None, :, :], s, NEG)

        m_prev = m_s[...]
        m_new = jnp.maximum(m_prev, jnp.max(s, axis=-1))
        alpha = jnp.exp(m_prev - m_new)
        p = jnp.exp(s - m_new[:, :, None]) * cnt[None, :, :]
        l_s[...] = l_s[...] * alpha + jnp.sum(p, axis=-1)
        pv = lax.dot_general(
            p.astype(jnp.bfloat16), v, (((2,), (0,)), ((0,), (1,))),
            preferred_element_type=jnp.float32,
        )
        acc_s[...] = acc_s[...] * alpha[:, :, None] + pv
        m_s[...] = m_new

        @pl.when(c == NC - 1)
        def _():
            comm[0, 0:H, :, :] = acc_s[...]
            comm[0, H, :, 0:B] = m_s[...]
            comm[0, H + 1, :, 0:B] = l_s[...]

            rdma = pltpu.make_async_remote_copy(
                src_ref=comm.at[0],
                dst_ref=comm.at[1],
                send_sem=send_sem,
                recv_sem=recv_sem,
                device_id=(1 - my_x, my_y, my_z),
                device_id_type=pl.DeviceIdType.MESH,
            )
            rdma.start()
            rdma.wait()

            acc1 = comm[1, 0:H, :, :]
            m1 = comm[1, H, :, 0:B]
            l1 = comm[1, H + 1, :, 0:B]
            m0, l0, acc0 = m_s[...], l_s[...], acc_s[...]
            m_all = jnp.maximum(m0, m1)
            a0 = jnp.exp(m0 - m_all)
            a1 = jnp.exp(m1 - m_all)
            l_all = a0 * l0 + a1 * l1
            acc = a0[:, :, None] * acc0 + a1[:, :, None] * acc1
            o = acc / l_all[:, :, None]
            for h in range(H):
                out_ref[:, 0, h, :] = o[h, :, :]

    return pl.pallas_call(
        body,
        grid=(NC,),
        in_specs=[
            pl.BlockSpec((B, 1, H, D), lambda c: (0, 0, 0, 0)),
            pl.BlockSpec((CP, BS, H, D), lambda c: (c, 0, 0, 0)),
            pl.BlockSpec((CP, BS, H, D), lambda c: (c, 0, 0, 0)),
            pl.BlockSpec((B, T), lambda c: (0, c)),
        ],
        out_specs=pl.BlockSpec((B, 1, H, D), lambda c: (0, 0, 0, 0)),
        out_shape=jax.ShapeDtypeStruct((B, 1, H, D), jnp.float32),
        scratch_shapes=[
            pltpu.VMEM((H, B, D), jnp.float32),
            pltpu.VMEM((H, B), jnp.float32),
            pltpu.VMEM((H, B), jnp.float32),
            pltpu.VMEM((2, H + 2, B, D), jnp.float32),
            pltpu.SemaphoreType.DMA,
            pltpu.SemaphoreType.DMA,
        ],
        compiler_params=pltpu.CompilerParams(collective_id=0),
    )(Q, K, V, cnt_tok)


# device time: 39634 ns/iter; 4.3213x vs baseline; 4.3213x over previous
import jax
import jax.numpy as jnp
from jax import lax
from jax.experimental import pallas as pl
from jax.experimental.pallas import tpu as pltpu

B, H, D, BS = 8, 8, 128, 16
R = B * H
NP_LOCAL = 512
CP = 32
NC = NP_LOCAL // CP
T = CP * BS
NEG = -1e30


def kernel(Q, K, V, bt, lens):
    my_x = lax.axis_index("x")
    lo = my_x * NP_LOCAL

    nb = bt.shape[1]
    valid = (jnp.arange(nb)[None, :] < lens[:, None]) & (bt >= lo) & (bt < lo + NP_LOCAL)
    rel = bt - lo
    onehot = (rel[:, :, None] == jnp.arange(NP_LOCAL)[None, None, :]) & valid[:, :, None]
    counts = onehot.sum(axis=1).astype(jnp.float32)
    cnt_tok = jnp.repeat(counts, BS, axis=1)
    cnt_rows = jnp.repeat(cnt_tok, H, axis=0)

    def body(q_ref, k_ref, v_ref, c_ref, out_ref,
             acc_s, m_s, l_s, qbd_s, comm, send_sem, recv_sem):
        c = pl.program_id(0)
        my_xx = lax.axis_index("x")
        my_y = lax.axis_index("y")
        my_z = lax.axis_index("z")

        @pl.when(c == 0)
        def _():
            barrier = pltpu.get_barrier_semaphore()
            pl.semaphore_signal(
                barrier, inc=1,
                device_id=(1 - my_xx, my_y, my_z),
                device_id_type=pl.DeviceIdType.MESH,
            )
            pl.semaphore_wait(barrier, 1)
            m_s[...] = jnp.full((R, 1), NEG, jnp.float32)
            l_s[...] = jnp.zeros((R, 1), jnp.float32)
            acc_s[...] = jnp.zeros((R, D), jnp.float32)
            q = q_ref[:, 0, :, :]
            eye = (
                lax.broadcasted_iota(jnp.int32, (H, H), 0)
                == lax.broadcasted_iota(jnp.int32, (H, H), 1)
            ).astype(jnp.float32)
            qbd = (q[:, :, None, :] * eye[None, :, :, None]).reshape(R, H * D)
            qbd_s[...] = qbd.astype(jnp.bfloat16)

        kf = k_ref[...].reshape(T, H * D).astype(jnp.bfloat16)
        vf = v_ref[...].reshape(T, H * D).astype(jnp.bfloat16)
        cnt = c_ref[...]

        s = lax.dot_general(
            qbd_s[...], kf, (((1,), (1,)), ((), ())),
            preferred_element_type=jnp.float32,
        ) * (D ** -0.5)
        s = jnp.where(cnt > 0.0, s, NEG)

        m_prev = m_s[...]
        m_new = jnp.maximum(m_prev, jnp.max(s, axis=1, keepdims=True))
        alpha = jnp.exp(m_prev - m_new)
        p = jnp.exp(s - m_new) * cnt
        l_s[...] = l_s[...] * alpha + jnp.sum(p, axis=1, keepdims=True)
        pv = lax.dot_general(
            p.astype(jnp.bfloat16), vf, (((1,), (0,)), ((), ())),
            preferred_element_type=jnp.float32,
        )
        hmask = (
            lax.broadcasted_iota(jnp.int32, (R, H), 0) % H
            == lax.broadcasted_iota(jnp.int32, (R, H), 1)
        ).astype(jnp.float32)
        pv_own = jnp.sum(pv.reshape(R, H, D) * hmask[:, :, None], axis=1)
        acc_s[...] = acc_s[...] * alpha + pv_own
        m_s[...] = m_new

        @pl.when(c == NC - 1)
        def _():
            comm[0, 0] = acc_s[...]
            comm[0, 1, :, 0:1] = m_s[...]
            comm[0, 2, :, 0:1] = l_s[...]

            rdma = pltpu.make_async_remote_copy(
                src_ref=comm.at[0],
                dst_ref=comm.at[1],
                send_sem=send_sem,
                recv_sem=recv_sem,
                device_id=(1 - my_xx, my_y, my_z),
                device_id_type=pl.DeviceIdType.MESH,
            )
            rdma.start()
            rdma.wait()

            acc1 = comm[1, 0]
            m1 = comm[1, 1, :, 0:1]
            l1 = comm[1, 2, :, 0:1]
            m0, l0, acc0 = m_s[...], l_s[...], acc_s[...]
            m_all = jnp.maximum(m0, m1)
            a0 = jnp.exp(m0 - m_all)
            a1 = jnp.exp(m1 - m_all)
            l_all = a0 * l0 + a1 * l1
            o = (a0 * acc0 + a1 * acc1) / l_all
            out_ref[:, 0, :, :] = o.reshape(B, H, D)

    return pl.pallas_call(
        body,
        grid=(NC,),
        in_specs=[
            pl.BlockSpec((B, 1, H, D), lambda c: (0, 0, 0, 0)),
            pl.BlockSpec((CP, BS, H, D), lambda c: (c, 0, 0, 0)),
            pl.BlockSpec((CP, BS, H, D), lambda c: (c, 0, 0, 0)),
            pl.BlockSpec((R, T), lambda c: (0, c)),
        ],
        out_specs=pl.BlockSpec((B, 1, H, D), lambda c: (0, 0, 0, 0)),
        out_shape=jax.ShapeDtypeStruct((B, 1, H, D), jnp.float32),
        scratch_shapes=[
            pltpu.VMEM((R, D), jnp.float32),
            pltpu.VMEM((R, 1), jnp.float32),
            pltpu.VMEM((R, 1), jnp.float32),
            pltpu.VMEM((R, H * D), jnp.bfloat16),
            pltpu.VMEM((2, 3, R, D), jnp.float32),
            pltpu.SemaphoreType.DMA,
            pltpu.SemaphoreType.DMA,
        ],
        compiler_params=pltpu.CompilerParams(collective_id=0),
    )(Q, K, V, cnt_rows)


# device time: 19540 ns/iter; 8.7651x vs baseline; 2.0284x over previous
import jax
import jax.numpy as jnp
from jax import lax
from jax.experimental import pallas as pl
from jax.experimental.pallas import tpu as pltpu

B, H, D, BS = 8, 8, 128, 16
R = B * H
NP_SHARD = 512
NP_DEV = 64
CP = 16
NC = NP_DEV // CP
T = CP * BS
NEG = -1e30
N_PEERS = 6


def kernel(Q, K, V, bt, lens):
    my_x = lax.axis_index("x")
    my_y = lax.axis_index("y")
    my_z = lax.axis_index("z")
    idx8 = my_y * 4 + my_z
    lo = my_x * NP_SHARD + idx8 * NP_DEV

    nb = bt.shape[1]
    rel = bt - lo
    valid = (jnp.arange(nb)[None, :] < lens[:, None]) & (rel >= 0) & (rel < NP_DEV)
    onehot = (rel[:, :, None] == jnp.arange(NP_DEV)[None, None, :]) & valid[:, :, None]
    counts = onehot.sum(axis=1).astype(jnp.float32)
    cnt_tok = jnp.repeat(counts, BS, axis=1)

    widx = jnp.reshape(idx8, (1,)).astype(jnp.int32)

    def body(w_ref, q_ref, k_ref, v_ref, c_ref, out_ref,
             acc_s, m_s, l_s, qbd_s, comm, send_sems, recv_sems):
        c = pl.program_id(0)
        x = lax.axis_index("x")
        y = lax.axis_index("y")
        z = lax.axis_index("z")

        peers_a = ((1 - x, y, z), (x, 1 - y, z), (1 - x, 1 - y, z))
        peers_b = tuple((x, y, (z + d) % 4) for d in (1, 2, 3))
        slots_a = (1, 2, 3)
        slots_b = (6, 5, 4)

        @pl.when(c == 0)
        def _():
            barrier = pltpu.get_barrier_semaphore()
            for peer in peers_a + peers_b:
                pl.semaphore_signal(
                    barrier, inc=1,
                    device_id=peer, device_id_type=pl.DeviceIdType.MESH,
                )
            pl.semaphore_wait(barrier, N_PEERS)
            m_s[...] = jnp.full((R, 1), NEG, jnp.float32)
            l_s[...] = jnp.zeros((R, 1), jnp.float32)
            acc_s[...] = jnp.zeros((R, D), jnp.float32)
            q = q_ref[:, 0, :, :]
            eye = (
                lax.broadcasted_iota(jnp.int32, (H, H), 0)
                == lax.broadcasted_iota(jnp.int32, (H, H), 1)
            ).astype(jnp.float32)
            qbd_s[...] = (q[:, :, None, :] * eye[None, :, :, None]).reshape(R, H * D)

        kf = k_ref[...].reshape(T, H * D)
        vf = v_ref[...].reshape(T, H * D)
        cb = c_ref[...]
        cnt = jnp.broadcast_to(cb[:, None, :], (B, H, T)).reshape(R, T)

        s = lax.dot_general(
            qbd_s[...], kf, (((1,), (1,)), ((), ())),
            preferred_element_type=jnp.float32,
        ) * (D ** -0.5)
        s = jnp.where(cnt > 0.0, s, NEG)

        m_prev = m_s[...]
        m_new = jnp.maximum(m_prev, jnp.max(s, axis=1, keepdims=True))
        alpha = jnp.exp(m_prev - m_new)
        p = jnp.exp(s - m_new) * cnt
        l_s[...] = l_s[...] * alpha + jnp.sum(p, axis=1, keepdims=True)
        pv = lax.dot_general(
            p, vf, (((1,), (0,)), ((), ())),
            preferred_element_type=jnp.float32,
        )
        hmask = (
            lax.broadcasted_iota(jnp.int32, (R, H), 0) % H
            == lax.broadcasted_iota(jnp.int32, (R, H), 1)
        ).astype(jnp.float32)
        pv_own = jnp.sum(pv.reshape(R, H, D) * hmask[:, :, None], axis=1)
        acc_s[...] = acc_s[...] * alpha + pv_own
        m_s[...] = m_new

        @pl.when(c == NC - 1)
        def _():
            def stage_partial():
                comm[0, 0:R, :] = acc_s[...].astype(jnp.bfloat16)
                comm[0, R:R + 1, 0:R] = jnp.transpose(
                    m_s[...].astype(jnp.bfloat16), (1, 0))
                comm[0, R + 1:R + 2, 0:R] = jnp.transpose(
                    l_s[...].astype(jnp.bfloat16), (1, 0))

            def merge(slot):
                acc1 = comm[slot, 0:R, :].astype(jnp.float32)
                m1 = jnp.transpose(comm[slot, R:R + 1, 0:R], (1, 0)).astype(jnp.float32)
                l1 = jnp.transpose(comm[slot, R + 1:R + 2, 0:R], (1, 0)).astype(jnp.float32)
                m0, l0, acc0 = m_s[...], l_s[...], acc_s[...]
                m_all = jnp.maximum(m0, m1)
                a0 = jnp.exp(m0 - m_all)
                a1 = jnp.exp(m1 - m_all)
                m_s[...] = m_all
                l_s[...] = a0 * l0 + a1 * l1
                acc_s[...] = a0 * acc0 + a1 * acc1

            for peers, slots in ((peers_a, slots_a), (peers_b, slots_b)):
                stage_partial()
                rdmas = []
                for peer, slot in zip(peers, slots):
                    rdma = pltpu.make_async_remote_copy(
                        src_ref=comm.at[0],
                        dst_ref=comm.at[slot],
                        send_sem=send_sems.at[slot - 1],
                        recv_sem=recv_sems.at[slot - 1],
                        device_id=peer,
                        device_id_type=pl.DeviceIdType.MESH,
                    )
                    rdma.start()
                    rdmas.append(rdma)
                for rdma, (_, slot) in zip(rdmas, zip(peers, slots)):
                    rdma.wait()
                    merge(slot)

            out_ref[:, 0, :, :] = (acc_s[...] / l_s[...]).reshape(B, H, D)

    grid_spec = pltpu.PrefetchScalarGridSpec(
        num_scalar_prefetch=1,
        grid=(NC,),
        in_specs=[
            pl.BlockSpec((B, 1, H, D), lambda c, w: (0, 0, 0, 0)),
            pl.BlockSpec((CP, BS, H, D), lambda c, w: (w[0] * NC + c, 0, 0, 0)),
            pl.BlockSpec((CP, BS, H, D), lambda c, w: (w[0] * NC + c, 0, 0, 0)),
            pl.BlockSpec((B, T), lambda c, w: (0, c)),
        ],
        out_specs=pl.BlockSpec((B, 1, H, D), lambda c, w: (0, 0, 0, 0)),
        scratch_shapes=[
            pltpu.VMEM((R, D), jnp.float32),
            pltpu.VMEM((R, 1), jnp.float32),
            pltpu.VMEM((R, 1), jnp.float32),
            pltpu.VMEM((R, H * D), jnp.float32),
            pltpu.VMEM((1 + N_PEERS, R + 2, D), jnp.bfloat16),
            pltpu.SemaphoreType.DMA((N_PEERS,)),
            pltpu.SemaphoreType.DMA((N_PEERS,)),
        ],
    )

    return pl.pallas_call(
        body,
        grid_spec=grid_spec,
        out_shape=jax.ShapeDtypeStruct((B, 1, H, D), jnp.float32),
        compiler_params=pltpu.CompilerParams(collective_id=0),
    )(widx, Q, K, V, cnt_tok)


# device time: 18559 ns/iter; 9.2284x vs baseline; 1.0529x over previous
import jax
import jax.numpy as jnp
from jax import lax
from jax.experimental import pallas as pl
from jax.experimental.pallas import tpu as pltpu

B, H, D, BS = 8, 8, 128, 16
R = B * H
NP_SHARD = 512
NP_DEV = 64
CP = 64
NC = NP_DEV // CP
T = CP * BS
NEG = -1e30
N_PEERS = 6


def kernel(Q, K, V, bt, lens):
    my_x = lax.axis_index("x")
    my_y = lax.axis_index("y")
    my_z = lax.axis_index("z")
    idx8 = my_y * 4 + my_z
    lo = my_x * NP_SHARD + idx8 * NP_DEV

    nb = bt.shape[1]
    rel = bt - lo
    valid = (jnp.arange(nb)[None, :] < lens[:, None]) & (rel >= 0) & (rel < NP_DEV)
    onehot = (rel[:, :, None] == jnp.arange(NP_DEV)[None, None, :]) & valid[:, :, None]
    counts = onehot.sum(axis=1).astype(jnp.float32)
    cnt_tok = jnp.repeat(counts, BS, axis=1)

    widx = jnp.reshape(idx8, (1,)).astype(jnp.int32)

    def body(w_ref, q_ref, k_ref, v_ref, c_ref, out_ref,
             acc_s, m_s, l_s, qbd_s, comm, send_sems, recv_sems):
        c = pl.program_id(0)
        x = lax.axis_index("x")
        y = lax.axis_index("y")
        z = lax.axis_index("z")

        peers_a = ((1 - x, y, z), (x, 1 - y, z), (1 - x, 1 - y, z))
        peers_b = tuple((x, y, (z + d) % 4) for d in (1, 2, 3))
        slots_a = (1, 2, 3)
        slots_b = (6, 5, 4)

        @pl.when(c == 0)
        def _():
            barrier = pltpu.get_barrier_semaphore()
            for peer in peers_a + peers_b:
                pl.semaphore_signal(
                    barrier, inc=1,
                    device_id=peer, device_id_type=pl.DeviceIdType.MESH,
                )
            pl.semaphore_wait(barrier, N_PEERS)
            m_s[...] = jnp.full((R, 1), NEG, jnp.float32)
            l_s[...] = jnp.zeros((R, 1), jnp.float32)
            acc_s[...] = jnp.zeros((R, D), jnp.float32)
            q = q_ref[:, 0, :, :]
            eye = (
                lax.broadcasted_iota(jnp.int32, (H, H), 0)
                == lax.broadcasted_iota(jnp.int32, (H, H), 1)
            ).astype(jnp.float32)
            qbd_s[...] = (q[:, :, None, :] * eye[None, :, :, None]).reshape(R, H * D)

        kf = k_ref[...].reshape(T, H * D)
        vf = v_ref[...].reshape(T, H * D)
        cb = c_ref[...]
        cnt = jnp.broadcast_to(cb[:, None, :], (B, H, T)).reshape(R, T)

        s = lax.dot_general(
            qbd_s[...], kf, (((1,), (1,)), ((), ())),
            preferred_element_type=jnp.float32,
        ) * (D ** -0.5)
        s = jnp.where(cnt > 0.0, s, NEG)

        m_prev = m_s[...]
        m_new = jnp.maximum(m_prev, jnp.max(s, axis=1, keepdims=True))
        alpha = jnp.exp(m_prev - m_new)
        p = jnp.exp(s - m_new) * cnt
        l_s[...] = l_s[...] * alpha + jnp.sum(p, axis=1, keepdims=True)
        pv = lax.dot_general(
            p, vf, (((1,), (0,)), ((), ())),
            preferred_element_type=jnp.float32,
        )
        hmask = (
            lax.broadcasted_iota(jnp.int32, (R, H), 0) % H
            == lax.broadcasted_iota(jnp.int32, (R, H), 1)
        ).astype(jnp.float32)
        pv_own = jnp.sum(pv.reshape(R, H, D) * hmask[:, :, None], axis=1)
        acc_s[...] = acc_s[...] * alpha + pv_own
        m_s[...] = m_new

        @pl.when(c == NC - 1)
        def _():
            def stage_partial():
                comm[0, 0:R, :] = acc_s[...].astype(jnp.bfloat16)
                comm[0, R:R + 1, 0:R] = jnp.transpose(
                    m_s[...].astype(jnp.bfloat16), (1, 0))
                comm[0, R + 1:R + 2, 0:R] = jnp.transpose(
                    l_s[...].astype(jnp.bfloat16), (1, 0))

            def merge(slot):
                acc1 = comm[slot, 0:R, :].astype(jnp.float32)
                m1 = jnp.transpose(comm[slot, R:R + 1, 0:R], (1, 0)).astype(jnp.float32)
                l1 = jnp.transpose(comm[slot, R + 1:R + 2, 0:R], (1, 0)).astype(jnp.float32)
                m0, l0, acc0 = m_s[...], l_s[...], acc_s[...]
                m_all = jnp.maximum(m0, m1)
                a0 = jnp.exp(m0 - m_all)
                a1 = jnp.exp(m1 - m_all)
                m_s[...] = m_all
                l_s[...] = a0 * l0 + a1 * l1
                acc_s[...] = a0 * acc0 + a1 * acc1

            for peers, slots in ((peers_a, slots_a), (peers_b, slots_b)):
                stage_partial()
                rdmas = []
                for peer, slot in zip(peers, slots):
                    rdma = pltpu.make_async_remote_copy(
                        src_ref=comm.at[0],
                        dst_ref=comm.at[slot],
                        send_sem=send_sems.at[slot - 1],
                        recv_sem=recv_sems.at[slot - 1],
                        device_id=peer,
                        device_id_type=pl.DeviceIdType.MESH,
                    )
                    rdma.start()
                    rdmas.append(rdma)
                for rdma, (_, slot) in zip(rdmas, zip(peers, slots)):
                    rdma.wait()
                    merge(slot)

            out_ref[:, 0, :, :] = (acc_s[...] / l_s[...]).reshape(B, H, D)

    grid_spec = pltpu.PrefetchScalarGridSpec(
        num_scalar_prefetch=1,
        grid=(NC,),
        in_specs=[
            pl.BlockSpec((B, 1, H, D), lambda c, w: (0, 0, 0, 0)),
            pl.BlockSpec((CP, BS, H, D), lambda c, w: (w[0] * NC + c, 0, 0, 0)),
            pl.BlockSpec((CP, BS, H, D), lambda c, w: (w[0] * NC + c, 0, 0, 0)),
            pl.BlockSpec((B, T), lambda c, w: (0, c)),
        ],
        out_specs=pl.BlockSpec((B, 1, H, D), lambda c, w: (0, 0, 0, 0)),
        scratch_shapes=[
            pltpu.VMEM((R, D), jnp.float32),
            pltpu.VMEM((R, 1), jnp.float32),
            pltpu.VMEM((R, 1), jnp.float32),
            pltpu.VMEM((R, H * D), jnp.float32),
            pltpu.VMEM((1 + N_PEERS, R + 2, D), jnp.bfloat16),
            pltpu.SemaphoreType.DMA((N_PEERS,)),
            pltpu.SemaphoreType.DMA((N_PEERS,)),
        ],
    )

    return pl.pallas_call(
        body,
        grid_spec=grid_spec,
        out_shape=jax.ShapeDtypeStruct((B, 1, H, D), jnp.float32),
        compiler_params=pltpu.CompilerParams(collective_id=0),
    )(widx, Q, K, V, cnt_tok)


# device time: 18136 ns/iter; 9.4436x vs baseline; 1.0233x over previous
import jax
import jax.numpy as jnp
from jax import lax
from jax.experimental import pallas as pl
from jax.experimental.pallas import tpu as pltpu

B, H, D, BS = 8, 8, 128, 16
R = B * H
NP_SHARD = 512
NP_DEV = 64
CP = 32
NC = NP_DEV // CP
T = CP * BS
NEG = -1e30
N_PEERS = 6


def kernel(Q, K, V, bt, lens):
    my_x = lax.axis_index("x")
    my_y = lax.axis_index("y")
    my_z = lax.axis_index("z")
    idx8 = my_y * 4 + my_z
    lo = my_x * NP_SHARD + idx8 * NP_DEV

    nb = bt.shape[1]
    rel = bt - lo
    valid = (jnp.arange(nb)[None, :] < lens[:, None]) & (rel >= 0) & (rel < NP_DEV)
    onehot = (rel[:, :, None] == jnp.arange(NP_DEV)[None, None, :]) & valid[:, :, None]
    counts = onehot.sum(axis=1).astype(jnp.float32)
    cnt_tok = jnp.repeat(counts, BS, axis=1)

    widx = jnp.reshape(idx8, (1,)).astype(jnp.int32)

    def body(w_ref, q_ref, k_ref, v_ref, c_ref, out_ref,
             acc_s, m_s, l_s, qbd_s, comm, send_sems, recv_sems):
        c = pl.program_id(0)
        x = lax.axis_index("x")
        y = lax.axis_index("y")
        z = lax.axis_index("z")

        peers_a = ((1 - x, y, z), (x, 1 - y, z), (1 - x, 1 - y, z))
        peers_b = tuple((x, y, (z + d) % 4) for d in (1, 2, 3))
        slots_a = (1, 2, 3)
        slots_b = (6, 5, 4)

        @pl.when(c == 0)
        def _():
            barrier = pltpu.get_barrier_semaphore()
            for peer in peers_a + peers_b:
                pl.semaphore_signal(
                    barrier, inc=1,
                    device_id=peer, device_id_type=pl.DeviceIdType.MESH,
                )
            pl.semaphore_wait(barrier, N_PEERS)
            m_s[...] = jnp.full((R, 1), NEG, jnp.float32)
            l_s[...] = jnp.zeros((R, 1), jnp.float32)
            acc_s[...] = jnp.zeros((R, D), jnp.float32)
            q = q_ref[:, 0, :, :]
            eye = (
                lax.broadcasted_iota(jnp.int32, (H, H), 0)
                == lax.broadcasted_iota(jnp.int32, (H, H), 1)
            ).astype(jnp.float32)
            qbd_s[...] = (q[:, :, None, :] * eye[None, :, :, None]).reshape(R, H * D)

        kf = k_ref[...].reshape(T, H * D)
        vf = v_ref[...].reshape(T, H * D)
        cb = c_ref[...]
        cnt = jnp.broadcast_to(cb[:, None, :], (B, H, T)).reshape(R, T)

        s = lax.dot_general(
            qbd_s[...], kf, (((1,), (1,)), ((), ())),
            preferred_element_type=jnp.float32,
        ) * (D ** -0.5)
        s = jnp.where(cnt > 0.0, s, NEG)

        m_prev = m_s[...]
        m_new = jnp.maximum(m_prev, jnp.max(s, axis=1, keepdims=True))
        alpha = jnp.exp(m_prev - m_new)
        p = jnp.exp(s - m_new) * cnt
        l_s[...] = l_s[...] * alpha + jnp.sum(p, axis=1, keepdims=True)
        pv = lax.dot_general(
            p, vf, (((1,), (0,)), ((), ())),
            preferred_element_type=jnp.float32,
        )
        hmask = (
            lax.broadcasted_iota(jnp.int32, (R, H), 0) % H
            == lax.broadcasted_iota(jnp.int32, (R, H), 1)
        ).astype(jnp.float32)
        pv_own = jnp.sum(pv.reshape(R, H, D) * hmask[:, :, None], axis=1)
        acc_s[...] = acc_s[...] * alpha + pv_own
        m_s[...] = m_new

        @pl.when(c == NC - 1)
        def _():
            def stage_partial():
                comm[0, 0:R, :] = acc_s[...].astype(jnp.bfloat16)
                comm[0, R:R + 1, 0:R] = jnp.transpose(
                    m_s[...].astype(jnp.bfloat16), (1, 0))
                comm[0, R + 1:R + 2, 0:R] = jnp.transpose(
                    l_s[...].astype(jnp.bfloat16), (1, 0))

            def merge(slot):
                acc1 = comm[slot, 0:R, :].astype(jnp.float32)
                m1 = jnp.transpose(comm[slot, R:R + 1, 0:R], (1, 0)).astype(jnp.float32)
                l1 = jnp.transpose(comm[slot, R + 1:R + 2, 0:R], (1, 0)).astype(jnp.float32)
                m0, l0, acc0 = m_s[...], l_s[...], acc_s[...]
                m_all = jnp.maximum(m0, m1)
                a0 = jnp.exp(m0 - m_all)
                a1 = jnp.exp(m1 - m_all)
                m_s[...] = m_all
                l_s[...] = a0 * l0 + a1 * l1
                acc_s[...] = a0 * acc0 + a1 * acc1

            for peers, slots in ((peers_a, slots_a), (peers_b, slots_b)):
                stage_partial()
                rdmas = []
                for peer, slot in zip(peers, slots):
                    rdma = pltpu.make_async_remote_copy(
                        src_ref=comm.at[0],
                        dst_ref=comm.at[slot],
                        send_sem=send_sems.at[slot - 1],
                        recv_sem=recv_sems.at[slot - 1],
                        device_id=peer,
                        device_id_type=pl.DeviceIdType.MESH,
                    )
                    rdma.start()
                    rdmas.append(rdma)
                for rdma, (_, slot) in zip(rdmas, zip(peers, slots)):
                    rdma.wait()
                    merge(slot)

            out_ref[:, 0, :, :] = (acc_s[...] / l_s[...]).reshape(B, H, D)

    grid_spec = pltpu.PrefetchScalarGridSpec(
        num_scalar_prefetch=1,
        grid=(NC,),
        in_specs=[
            pl.BlockSpec((B, 1, H, D), lambda c, w: (0, 0, 0, 0)),
            pl.BlockSpec((CP, BS, H, D), lambda c, w: (w[0] * NC + c, 0, 0, 0)),
            pl.BlockSpec((CP, BS, H, D), lambda c, w: (w[0] * NC + c, 0, 0, 0)),
            pl.BlockSpec((B, T), lambda c, w: (0, c)),
        ],
        out_specs=pl.BlockSpec((B, 1, H, D), lambda c, w: (0, 0, 0, 0)),
        scratch_shapes=[
            pltpu.VMEM((R, D), jnp.float32),
            pltpu.VMEM((R, 1), jnp.float32),
            pltpu.VMEM((R, 1), jnp.float32),
            pltpu.VMEM((R, H * D), jnp.float32),
            pltpu.VMEM((1 + N_PEERS, R + 2, D), jnp.bfloat16),
            pltpu.SemaphoreType.DMA((N_PEERS,)),
            pltpu.SemaphoreType.DMA((N_PEERS,)),
        ],
    )

    return pl.pallas_call(
        body,
        grid_spec=grid_spec,
        out_shape=jax.ShapeDtypeStruct((B, 1, H, D), jnp.float32),
        compiler_params=pltpu.CompilerParams(collective_id=0),
    )(widx, Q, K, V, cnt_tok)
